# Initial kernel scaffold; baseline (speedup 1.0000x reference)
import jax
import jax.numpy as jnp
from jax.experimental import pallas as pl


def kernel(text, table):
    raise NotImplementedError("write your pallas kernel here")


# speedup vs baseline: 1.5017x; 1.5017x over previous
import functools

import jax
import jax.numpy as jnp
from jax import lax
from jax.experimental import pallas as pl
from jax.experimental.pallas import tpu as pltpu
from jax.experimental.pallas import tpu_sc as plsc

NC, NS = 2, 16
NW = NC * NS
CHUNK = 128
K = 8
EMBED = 32

BATCH, HIST = 4096, 200
TOTAL = BATCH * HIST
N_CHUNKS = TOTAL // CHUNK
CPW = N_CHUNKS // NW
G = CPW // K
M = (G - 2) // 2
assert CPW % K == 0 and G >= 4


def _make_gather():
  mesh = plsc.VectorSubcoreMesh(core_axis_name="c", subcore_axis_name="s")

  @functools.partial(
      pl.kernel,
      out_type=jax.ShapeDtypeStruct((N_CHUNKS, CHUNK, EMBED), jnp.float32),
      mesh=mesh,
      scratch_types=[
          pltpu.VMEM((2, K, CHUNK), jnp.int32),
          pltpu.VMEM((2, K, CHUNK, EMBED), jnp.float32),
          pltpu.SemaphoreType.DMA,
          pltpu.SemaphoreType.DMA,
      ],
      compiler_params=pltpu.CompilerParams(use_tc_tiling_on_sc=False),
  )
  def gather_kernel(idx_hbm, table_hbm, out_hbm, idx_v, rows_v, idx_sem,
                    gat_sem):
    wid = lax.axis_index("s") * NC + lax.axis_index("c")
    c0 = wid * CPW

    def idx_copy(g, slot):
      return pltpu.make_async_copy(
          idx_hbm.at[pl.ds(c0 + g * K, K)], idx_v.at[slot], idx_sem)

    def gat(slot, j):
      return pltpu.make_async_copy(
          table_hbm.at[idx_v.at[slot, j]], rows_v.at[slot, j], gat_sem)

    def fire(slot):
      for j in range(K):
        gat(slot, j).start()

    def drain(slot):
      for j in range(K):
        gat(slot, j).wait()

    def out_copy(g, slot):
      pltpu.sync_copy(rows_v.at[slot], out_hbm.at[pl.ds(c0 + g * K, K)])

    idx_copy(0, 0).start()
    idx_copy(0, 0).wait()
    fire(0)
    idx_copy(1, 1).start()

    def pair(p, carry):
      for s in range(2):
        g = 2 * p + s
        idx_copy(g + 1, 1 - s).wait()
        fire(1 - s)
        drain(s)
        idx_copy(g + 2, s).start()
        out_copy(g, s)
      return carry

    lax.fori_loop(0, M, pair, 0)

    for g in range(2 * M, G):
      if g + 1 < G:
        idx_copy(g + 1, (g + 1) % 2).wait()
        fire((g + 1) % 2)
      drain(g % 2)
      if g + 2 < G:
        idx_copy(g + 2, g % 2).start()
      out_copy(g, g % 2)

  return gather_kernel


_gather = _make_gather()


@jax.jit
def kernel(text, table):
  idx = text.reshape(N_CHUNKS, CHUNK).astype(jnp.int32)
  out = _gather(idx, table)
  return out.reshape(BATCH, HIST, EMBED)


# speedup vs baseline: 1.5018x; 1.0001x over previous
import functools

import jax
import jax.numpy as jnp
from jax import lax
from jax.experimental import pallas as pl
from jax.experimental.pallas import tpu as pltpu
from jax.experimental.pallas import tpu_sc as plsc

NC, NS = 2, 16
NW = NC * NS
GRP = 1024
EMBED = 32

BATCH, HIST = 4096, 200
TOTAL = BATCH * HIST
N_GROUPS = TOTAL // GRP
G = N_GROUPS // NW
assert TOTAL % GRP == 0 and N_GROUPS % NW == 0 and (G - 4) % 3 == 0


def _make_gather():
  mesh = plsc.VectorSubcoreMesh(core_axis_name="c", subcore_axis_name="s")

  @functools.partial(
      pl.kernel,
      out_type=jax.ShapeDtypeStruct((TOTAL, EMBED), jnp.float32),
      mesh=mesh,
      scratch_types=[
          pltpu.VMEM((3, GRP), jnp.int32),
          pltpu.VMEM((3, GRP, EMBED), jnp.float32),
          pltpu.SemaphoreType.DMA,
          pltpu.SemaphoreType.DMA,
          pltpu.SemaphoreType.DMA,
      ],
      compiler_params=pltpu.CompilerParams(use_tc_tiling_on_sc=False),
  )
  def gather_kernel(idx_hbm, table_hbm, out_hbm, idx_v, rows_v, idx_sem,
                    gat_sem, out_sem):
    wid = lax.axis_index("s") * NC + lax.axis_index("c")
    g0 = wid * G

    def idx_copy(g, slot):
      return pltpu.make_async_copy(
          idx_hbm.at[pl.ds((g0 + g) * GRP, GRP)], idx_v.at[slot], idx_sem)

    def gat(slot):
      return pltpu.make_async_copy(
          table_hbm.at[idx_v.at[slot]], rows_v.at[slot], gat_sem)

    def out_copy(g, slot):
      return pltpu.make_async_copy(
          rows_v.at[slot], out_hbm.at[pl.ds((g0 + g) * GRP, GRP)], out_sem)

    def step(g, s, have_ow, have_f, have_il):
      if have_ow:
        out_copy(g - 2, (g - 2) % 3).wait()
      if have_f:
        idx_copy(g + 1, (s + 1) % 3).wait()
        gat((s + 1) % 3).start()
      if have_il:
        idx_copy(g + 2, (s + 2) % 3).start()
      gat(s).wait()
      out_copy(g, s).start()

    idx_copy(0, 0).start()
    idx_copy(0, 0).wait()
    gat(0).start()
    idx_copy(1, 1).start()
    step(0, 0, False, True, True)
    step(1, 1, False, True, True)

    def trip(p, carry):
      for s in range(3):
        g = 3 * p + 2 + s
        step(g, (2 + s) % 3, True, True, True)
      return carry

    lax.fori_loop(0, (G - 4) // 3, trip, 0)

    for g in range(G - 2, G):
      step(g, g % 3, True, g + 1 < G, g + 2 < G)
    out_copy(G - 2, (G - 2) % 3).wait()
    out_copy(G - 1, (G - 1) % 3).wait()

  return gather_kernel


_gather = _make_gather()


@jax.jit
def kernel(text, table):
  idx = text.reshape(TOTAL).astype(jnp.int32)
  out = _gather(idx, table)
  return out.reshape(BATCH, HIST, EMBED)
